# baseline (speedup 1.0000x reference)
import functools

import jax
import jax.numpy as jnp
from jax import lax
from jax.experimental import pallas as pl
from jax.experimental.pallas import tpu as pltpu
from jax.sharding import PartitionSpec as P

MIN_T = 2
MAX_T = 980
K = 1000
ROWS_PER_BLOCK = 256


def _threefry_bits(idx_u32):
    ks = (jnp.uint32(0), jnp.uint32(1), jnp.uint32(0x1BD11BDB))
    rotations = ((13, 15, 26, 6), (17, 29, 16, 24))
    x0 = jnp.zeros_like(idx_u32)
    x1 = idx_u32 + jnp.uint32(1)
    for i in range(5):
        for r in rotations[i % 2]:
            x0 = x0 + x1
            x1 = ((x1 << jnp.uint32(r)) | (x1 >> jnp.uint32(32 - r))) ^ x0
        x0 = x0 + ks[(i + 1) % 3]
        x1 = x1 + ks[(i + 2) % 3] + jnp.uint32(i + 1)
    return x0 ^ x1


def _sample_block(base_ref, c_ref, o_ref, *, rows):
    x = c_ref[...]
    col = lax.broadcasted_iota(jnp.int32, (rows, K), 1)
    masked = (col < MIN_T) | (col >= MAX_T)
    logits = jnp.where(masked, x - 1e8, x)
    m = jnp.max(logits, axis=1, keepdims=True)
    e = jnp.exp(logits - m)
    s = jnp.sum(e, axis=1, keepdims=True)
    p = e / s + 1e-8
    lp = jnp.log(p)

    row = lax.broadcasted_iota(jnp.int32, (rows, K), 0)
    base = (base_ref[0] + pl.program_id(0) * rows) * K
    idx = (base + row * K + col).astype(jnp.uint32)
    bits = _threefry_bits(idx)

    fbits = (bits >> jnp.uint32(9)) | jnp.uint32(0x3F800000)
    f = lax.bitcast_convert_type(fbits, jnp.float32) - jnp.float32(1.0)
    tiny = jnp.float32(1.1754944e-38)
    u = f * (jnp.float32(1.0) - tiny) + tiny
    u = jnp.maximum(tiny, u)
    g = -jnp.log(-jnp.log(u))

    o_ref[...] = jnp.argmax(lp + g, axis=1).astype(jnp.int32)


def _sample_rows(x, base_row):
    local_rows = x.shape[0]
    grid = local_rows // ROWS_PER_BLOCK
    return pl.pallas_call(
        functools.partial(_sample_block, rows=ROWS_PER_BLOCK),
        grid=(grid,),
        in_specs=[
            pl.BlockSpec(memory_space=pltpu.SMEM),
            pl.BlockSpec((ROWS_PER_BLOCK, K), lambda i: (i, 0)),
        ],
        out_specs=pl.BlockSpec((ROWS_PER_BLOCK,), lambda i: (i,)),
        out_shape=jax.ShapeDtypeStruct((local_rows,), jnp.int32),
    )(base_row.reshape(1).astype(jnp.int32), x)


@jax.jit
def kernel(c):
    n, l, k = c.shape
    rows = n * l
    x = c.reshape(rows, k)

    ndev = len(jax.devices())
    while ndev > 1 and rows % (ndev * ROWS_PER_BLOCK):
        ndev -= 1
    if ndev > 1:
        mesh = jax.make_mesh((ndev,), ("d",))
        sh = jax.sharding.NamedSharding(mesh, P("d", None))
        chunks = 4
        while rows % (chunks * ndev * ROWS_PER_BLOCK):
            chunks //= 2
        chunk = rows // chunks
        local = chunk // ndev
        outs = []
        for i in range(chunks):
            xi = lax.slice(x, (i * chunk, 0), ((i + 1) * chunk, k))
            xi = jax.reshard(xi, sh)

            def shard_fn(xs, i=i):
                base = i * chunk + lax.axis_index("d") * local
                return _sample_rows(xs, base)

            outs.append(
                jax.shard_map(
                    shard_fn,
                    mesh=mesh,
                    in_specs=P("d", None),
                    out_specs=P("d"),
                    check_vma=False,
                )(xi)
            )
        out = jnp.concatenate(outs)
    else:
        out = _sample_rows(x, jnp.int32(0))
    return out.reshape(n, l).astype(jnp.int64)


# speedup vs baseline: 1.5498x; 1.5498x over previous
import functools

import jax
import jax.numpy as jnp
from jax import lax
from jax.experimental import pallas as pl
from jax.experimental.pallas import tpu as pltpu
from jax.sharding import PartitionSpec as P

MIN_T = 2
MAX_T = 980
K = 1000
ROWS_PER_BLOCK = 256


def _threefry_bits(idx_u32):
    ks = (jnp.uint32(0), jnp.uint32(1), jnp.uint32(0x1BD11BDB))
    rotations = ((13, 15, 26, 6), (17, 29, 16, 24))
    x0 = jnp.zeros_like(idx_u32)
    x1 = idx_u32 + jnp.uint32(1)
    for i in range(5):
        for r in rotations[i % 2]:
            x0 = x0 + x1
            x1 = ((x1 << jnp.uint32(r)) | (x1 >> jnp.uint32(32 - r))) ^ x0
        x0 = x0 + ks[(i + 1) % 3]
        x1 = x1 + ks[(i + 2) % 3] + jnp.uint32(i + 1)
    return x0 ^ x1


def _sample_block(base_ref, c_ref, o_ref, *, rows):
    x = c_ref[...]
    col = lax.broadcasted_iota(jnp.int32, (rows, K), 1)
    masked = (col < MIN_T) | (col >= MAX_T)
    logits = jnp.where(masked, x - 1e8, x)
    m = jnp.max(logits, axis=1, keepdims=True)
    e = jnp.exp(logits - m)
    s = jnp.sum(e, axis=1, keepdims=True)
    p = e / s + 1e-8
    lp = jnp.log(p)

    row = lax.broadcasted_iota(jnp.int32, (rows, K), 0)
    base = (base_ref[0] + pl.program_id(0) * rows) * K
    idx = (base + row * K + col).astype(jnp.uint32)
    bits = _threefry_bits(idx)

    fbits = (bits >> jnp.uint32(9)) | jnp.uint32(0x3F800000)
    f = lax.bitcast_convert_type(fbits, jnp.float32) - jnp.float32(1.0)
    u = f + jnp.float32(1.1754944e-38)
    g = -jnp.log(-jnp.log(u))

    o_ref[...] = jnp.argmax(lp + g, axis=1).astype(jnp.int32)


def _sample_rows(x, base_row):
    local_rows = x.shape[0]
    grid = local_rows // ROWS_PER_BLOCK
    return pl.pallas_call(
        functools.partial(_sample_block, rows=ROWS_PER_BLOCK),
        grid=(grid,),
        in_specs=[
            pl.BlockSpec(memory_space=pltpu.SMEM),
            pl.BlockSpec((ROWS_PER_BLOCK, K), lambda i: (i, 0)),
        ],
        out_specs=pl.BlockSpec((ROWS_PER_BLOCK,), lambda i: (i,)),
        out_shape=jax.ShapeDtypeStruct((local_rows,), jnp.int32),
    )(base_row.reshape(1).astype(jnp.int32), x)


@jax.jit
def kernel(c):
    n, l, k = c.shape
    rows = n * l
    x = c.reshape(rows, k)

    ndev = len(jax.devices())
    while ndev > 1 and rows % (ndev * ROWS_PER_BLOCK):
        ndev -= 1
    if ndev > 1:
        mesh = jax.make_mesh((ndev,), ("d",))
        local = rows // ndev
        x = jax.reshard(x, jax.sharding.NamedSharding(mesh, P("d", None)))

        def shard_fn(xs):
            base = lax.axis_index("d") * local
            return _sample_rows(xs, base)

        out = jax.shard_map(
            shard_fn,
            mesh=mesh,
            in_specs=P("d", None),
            out_specs=P("d"),
            check_vma=False,
        )(x)
    else:
        out = _sample_rows(x, jnp.int32(0))
    return out.reshape(n, l).astype(jnp.int64)


# speedup vs baseline: 1.5963x; 1.0300x over previous
import functools

import jax
import jax.numpy as jnp
from jax import lax
from jax.experimental import pallas as pl
from jax.experimental.pallas import tpu as pltpu
from jax.sharding import PartitionSpec as P

MIN_T = 2
MAX_T = 980
K = 1000
ROWS_PER_BLOCK = 256


def _threefry_bits(idx_u32):
    ks = (jnp.uint32(0), jnp.uint32(1), jnp.uint32(0x1BD11BDB))
    rotations = ((13, 15, 26, 6), (17, 29, 16, 24))
    x0 = jnp.zeros_like(idx_u32)
    x1 = idx_u32 + jnp.uint32(1)
    for i in range(5):
        for r in rotations[i % 2]:
            x0 = x0 + x1
            x1 = ((x1 << jnp.uint32(r)) | (x1 >> jnp.uint32(32 - r))) ^ x0
        x0 = x0 + ks[(i + 1) % 3]
        x1 = x1 + ks[(i + 2) % 3] + jnp.uint32(i + 1)
    return x0 ^ x1


def _logp(x, rows):
    col = lax.broadcasted_iota(jnp.int32, (rows, K), 1)
    masked = (col < MIN_T) | (col >= MAX_T)
    logits = jnp.where(masked, x - 1e8, x)
    m = jnp.max(logits, axis=1, keepdims=True)
    e = jnp.exp(logits - m)
    s = jnp.sum(e, axis=1, keepdims=True)
    p = e / s + 1e-8
    return jnp.log(p)


def _gumbel(base_row, rows):
    row = lax.broadcasted_iota(jnp.int32, (rows, K), 0)
    col = lax.broadcasted_iota(jnp.int32, (rows, K), 1)
    idx = ((base_row + row) * K + col).astype(jnp.uint32)
    bits = _threefry_bits(idx)
    fbits = (bits >> jnp.uint32(9)) | jnp.uint32(0x3F800000)
    f = lax.bitcast_convert_type(fbits, jnp.float32) - jnp.float32(1.0)
    u = f + jnp.float32(1.1754944e-38)
    return -jnp.log(-jnp.log(u))


def _sample_block(base_ref, c_ref, o_ref, *, rows):
    lp = _logp(c_ref[...], rows)
    g = _gumbel(base_ref[0] + pl.program_id(0) * rows, rows)
    o_ref[...] = jnp.argmax(lp + g, axis=1).astype(jnp.int32)


def _sample_block_skip(scal_ref, c_ref, o_ref, *, rows, head_blocks):
    pid = pl.program_id(0)

    @pl.when((scal_ref[1] == 1) | (pid < head_blocks))
    def _():
        lp = _logp(c_ref[...], rows)
        g = _gumbel(scal_ref[0] + pid * rows, rows)
        o_ref[...] = jnp.argmax(lp + g, axis=1).astype(jnp.int32)


def _gen_block(flag_ref, o_ref, *, rows, base_row):
    pid = pl.program_id(0)

    @pl.when(flag_ref[0] == 1)
    def _():
        o_ref[...] = _gumbel(base_row + pid * rows, rows)

    @pl.when(flag_ref[0] == 0)
    def _():
        o_ref[...] = jnp.zeros((rows, K), jnp.float32)


def _consume_block(c_ref, g_ref, o_ref, *, rows):
    lp = _logp(c_ref[...], rows)
    o_ref[...] = jnp.argmax(lp + g_ref[...], axis=1).astype(jnp.int32)


def _sample_rows(x, base_row):
    local_rows = x.shape[0]
    grid = local_rows // ROWS_PER_BLOCK
    return pl.pallas_call(
        functools.partial(_sample_block, rows=ROWS_PER_BLOCK),
        grid=(grid,),
        in_specs=[
            pl.BlockSpec(memory_space=pltpu.SMEM),
            pl.BlockSpec((ROWS_PER_BLOCK, K), lambda i: (i, 0)),
        ],
        out_specs=pl.BlockSpec((ROWS_PER_BLOCK,), lambda i: (i,)),
        out_shape=jax.ShapeDtypeStruct((local_rows,), jnp.int32),
    )(base_row.reshape(1).astype(jnp.int32), x)


@jax.jit
def kernel(c):
    n, l, k = c.shape
    rows = n * l
    x = c.reshape(rows, k)

    ndev = len(jax.devices())
    while ndev > 1 and rows % (ndev * ROWS_PER_BLOCK):
        ndev -= 1
    if ndev == 2:
        mesh = jax.make_mesh((ndev,), ("d",))
        local = rows // ndev
        x = jax.reshard(x, jax.sharding.NamedSharding(mesh, P("d", None)))
        nblk = local // ROWS_PER_BLOCK
        tail_blocks = max(1, min(nblk - 1, round(0.28 * nblk)))
        head_blocks = nblk - tail_blocks
        tail = tail_blocks * ROWS_PER_BLOCK

        def shard_fn(xs):
            ax = lax.axis_index("d")
            g = pl.pallas_call(
                functools.partial(
                    _gen_block, rows=ROWS_PER_BLOCK, base_row=local - tail
                ),
                grid=(tail_blocks,),
                in_specs=[pl.BlockSpec(memory_space=pltpu.SMEM)],
                out_specs=pl.BlockSpec((ROWS_PER_BLOCK, K), lambda i: (i, 0)),
                out_shape=jax.ShapeDtypeStruct((tail, K), jnp.float32),
            )(ax.reshape(1).astype(jnp.int32))
            g_recv = lax.ppermute(g, "d", perm=[(1, 0)])
            out_a = pl.pallas_call(
                functools.partial(
                    _sample_block_skip,
                    rows=ROWS_PER_BLOCK,
                    head_blocks=head_blocks,
                ),
                grid=(nblk,),
                in_specs=[
                    pl.BlockSpec(memory_space=pltpu.SMEM),
                    pl.BlockSpec((ROWS_PER_BLOCK, K), lambda i: (i, 0)),
                ],
                out_specs=pl.BlockSpec((ROWS_PER_BLOCK,), lambda i: (i,)),
                out_shape=jax.ShapeDtypeStruct((local,), jnp.int32),
            )(jnp.stack([ax * local, ax]).astype(jnp.int32), xs)
            out_b = pl.pallas_call(
                functools.partial(_consume_block, rows=ROWS_PER_BLOCK),
                grid=(tail_blocks,),
                in_specs=[
                    pl.BlockSpec((ROWS_PER_BLOCK, K), lambda i: (i, 0)),
                    pl.BlockSpec((ROWS_PER_BLOCK, K), lambda i: (i, 0)),
                ],
                out_specs=pl.BlockSpec((ROWS_PER_BLOCK,), lambda i: (i,)),
                out_shape=jax.ShapeDtypeStruct((tail,), jnp.int32),
            )(lax.slice(xs, (local - tail, 0), (local, k)), g_recv)
            merged = jnp.concatenate([out_a[: local - tail], out_b])
            return jnp.where(ax == 0, merged, out_a)

        out = jax.shard_map(
            shard_fn,
            mesh=mesh,
            in_specs=P("d", None),
            out_specs=P("d"),
            check_vma=False,
        )(x)
    elif ndev > 1:
        mesh = jax.make_mesh((ndev,), ("d",))
        local = rows // ndev
        x = jax.reshard(x, jax.sharding.NamedSharding(mesh, P("d", None)))

        def shard_fn(xs):
            base = lax.axis_index("d") * local
            return _sample_rows(xs, base)

        out = jax.shard_map(
            shard_fn,
            mesh=mesh,
            in_specs=P("d", None),
            out_specs=P("d"),
            check_vma=False,
        )(x)
    else:
        out = _sample_rows(x, jnp.int32(0))
    return out.reshape(n, l).astype(jnp.int64)
